# Initial kernel scaffold; baseline (speedup 1.0000x reference)
import jax
import jax.numpy as jnp
from jax.experimental import pallas as pl


def kernel(changes_obs, changes_pred, bin_edges, bin_midpoints, bin_weights):
    raise NotImplementedError("write your pallas kernel here")


# speedup vs baseline: 1.2668x; 1.2668x over previous
import functools

import jax
import jax.numpy as jnp
from jax.experimental import pallas as pl
from jax.experimental.pallas import tpu as pltpu

NB = 32
LANES = 128


def _hist_loss_kernel(edges_ref, obs_ref, pred_ref, mids_ref, wts_ref,
                      loss_ref, p_obs_ref, p_pred_ref, *, batch):
    b = pl.program_id(0)

    def counts_of(x):
        parts = []
        for j in range(NB + 1):
            e = edges_ref[j]
            parts.append(jnp.sum((x >= e).astype(jnp.float32), axis=0,
                                 keepdims=True))
        parts.append(jnp.sum((x > edges_ref[NB]).astype(jnp.float32), axis=0,
                             keepdims=True))
        svec = jnp.concatenate(parts, axis=0)
        lower = svec[0:NB]
        upper = jnp.concatenate([svec[1:NB], svec[NB + 1:NB + 2]], axis=0)
        percol = lower - upper
        return jnp.sum(percol, axis=1, keepdims=True)

    c_obs = counts_of(obs_ref[0])
    c_pred = counts_of(pred_ref[0])

    t_obs = jnp.maximum(jnp.sum(c_obs), 1.0)
    t_pred = jnp.maximum(jnp.sum(c_pred), 1.0)
    p_obs = c_obs / t_obs
    p_pred = c_pred / t_pred
    p_obs_ref[0] = p_obs
    p_pred_ref[0] = p_pred

    r = jax.lax.broadcasted_iota(jnp.int32, (NB, NB), 0)
    c = jax.lax.broadcasted_iota(jnp.int32, (NB, NB), 1)
    tri = (c <= r).astype(jnp.float32)
    d = p_obs - p_pred
    cdf_d = jax.lax.dot_general(
        tri, d, (((1,), (0,)), ((), ())),
        preferred_element_type=jnp.float32)

    m = mids_ref[...]
    bw = jnp.concatenate([m[1:] - m[:-1], m[-1:] - m[-2:-1]], axis=0)
    contrib = jnp.sum(cdf_d * cdf_d * bw * wts_ref[...]) / batch

    @pl.when(b == 0)
    def _():
        loss_ref[0, 0] = 0.0

    loss_ref[0, 0] += contrib


@jax.jit
def kernel(changes_obs, changes_pred, bin_edges, bin_midpoints, bin_weights):
    B = changes_obs.shape[0]
    S = changes_obs.shape[1] * changes_obs.shape[2] // LANES
    obs = changes_obs.reshape(B, S, LANES)
    pred = changes_pred.reshape(B, S, LANES)
    mids = bin_midpoints.reshape(NB, 1)
    wts = bin_weights.reshape(NB, 1)

    grid = (B,)
    loss, p_obs, p_pred = pl.pallas_call(
        functools.partial(_hist_loss_kernel, batch=float(B)),
        grid=grid,
        in_specs=[
            pl.BlockSpec(memory_space=pltpu.SMEM),
            pl.BlockSpec((1, S, LANES), lambda b: (b, 0, 0)),
            pl.BlockSpec((1, S, LANES), lambda b: (b, 0, 0)),
            pl.BlockSpec((NB, 1), lambda b: (0, 0)),
            pl.BlockSpec((NB, 1), lambda b: (0, 0)),
        ],
        out_specs=[
            pl.BlockSpec(memory_space=pltpu.SMEM),
            pl.BlockSpec((1, NB, 1), lambda b: (b, 0, 0)),
            pl.BlockSpec((1, NB, 1), lambda b: (b, 0, 0)),
        ],
        out_shape=[
            jax.ShapeDtypeStruct((1, 1), jnp.float32),
            jax.ShapeDtypeStruct((B, NB, 1), jnp.float32),
            jax.ShapeDtypeStruct((B, NB, 1), jnp.float32),
        ],
    )(bin_edges, obs, pred, mids, wts)

    return (loss[0, 0], p_obs.reshape(B, NB), p_pred.reshape(B, NB))
